# Initial kernel scaffold; baseline (speedup 1.0000x reference)
import jax
import jax.numpy as jnp
from jax.experimental import pallas as pl


def kernel(queries, memory_bank):
    raise NotImplementedError("write your pallas kernel here")


# speedup vs baseline: 1.7585x; 1.7585x over previous
import functools

import jax
import jax.numpy as jnp
from jax.experimental import pallas as pl
from jax.experimental.pallas import tpu as pltpu

K_NN = 3
BLOCK_N = 1024


def _knn_step(q_ref, m_ref, vals_ref, idx_ref, rv_ref, ri_ref, *,
              block_n, n_total):
    i = pl.program_id(0)
    nsteps = pl.num_programs(0)

    @pl.when(i == 0)
    def _init():
        rv_ref[...] = jnp.full(rv_ref.shape, jnp.inf, jnp.float32)
        ri_ref[...] = jnp.zeros(ri_ref.shape, jnp.int32)

    q = q_ref[...]
    m = m_ref[...]
    qsq = jnp.sum(q * q, axis=1)
    msq = jnp.sum(m * m, axis=1)
    ab = jax.lax.dot_general(q, m, (((1,), (1,)), ((), ())),
                             preferred_element_type=jnp.float32)
    d2 = (qsq[:, None] + msq[None, :]) - 2.0 * ab
    dist = jnp.sqrt(jnp.maximum(d2, 1e-12))

    iota = jax.lax.broadcasted_iota(jnp.int32, dist.shape, 1)
    base = i * block_n

    r0 = rv_ref[0, :]
    r1 = rv_ref[1, :]
    r2 = rv_ref[2, :]
    j0 = ri_ref[0, :]
    j1 = ri_ref[1, :]
    j2 = ri_ref[2, :]

    work = dist
    for _ in range(K_NN):
        mval = jnp.min(work, axis=1)
        hit = work == mval[:, None]
        midx = jnp.min(jnp.where(hit, iota, n_total), axis=1)
        work = jnp.where(iota == midx[:, None], jnp.inf, work)
        gidx = midx + base
        b0 = mval < r0
        b1 = mval < r1
        b2 = mval < r2
        r0, r1, r2, j0, j1, j2 = (
            jnp.where(b0, mval, r0),
            jnp.where(b0, r0, jnp.where(b1, mval, r1)),
            jnp.where(b1, r1, jnp.where(b2, mval, r2)),
            jnp.where(b0, gidx, j0),
            jnp.where(b0, j0, jnp.where(b1, gidx, j1)),
            jnp.where(b1, j1, jnp.where(b2, gidx, j2)),
        )

    rv_ref[0, :] = r0
    rv_ref[1, :] = r1
    rv_ref[2, :] = r2
    ri_ref[0, :] = j0
    ri_ref[1, :] = j1
    ri_ref[2, :] = j2

    @pl.when(i == nsteps - 1)
    def _finish():
        vals_ref[...] = jnp.stack([r0, r1, r2], axis=1)
        idx_ref[...] = jnp.stack([j0, j1, j2], axis=1)


def kernel(queries, memory_bank):
    q_n, dim = queries.shape
    n_total, _ = memory_bank.shape
    block_n = min(BLOCK_N, n_total)
    grid = n_total // block_n

    vals, idx = pl.pallas_call(
        functools.partial(_knn_step, block_n=block_n, n_total=n_total),
        grid=(grid,),
        in_specs=[
            pl.BlockSpec((q_n, dim), lambda i: (0, 0)),
            pl.BlockSpec((block_n, dim), lambda i: (i, 0)),
        ],
        out_specs=[
            pl.BlockSpec((q_n, K_NN), lambda i: (0, 0)),
            pl.BlockSpec((q_n, K_NN), lambda i: (0, 0)),
        ],
        out_shape=[
            jax.ShapeDtypeStruct((q_n, K_NN), jnp.float32),
            jax.ShapeDtypeStruct((q_n, K_NN), jnp.int32),
        ],
        scratch_shapes=[
            pltpu.VMEM((8, q_n), jnp.float32),
            pltpu.VMEM((8, q_n), jnp.int32),
        ],
    )(queries, memory_bank)
    return vals, idx
